# baseline (speedup 1.0000x reference)
import functools

import jax
import jax.numpy as jnp
from jax import lax
from jax.experimental import pallas as pl
from jax.experimental.pallas import tpu as pltpu
from jax.experimental.pallas import tpu_sc as plsc

_N = 10000
_E = 320000
_NC = 2
_NS = 16
_CHUNK = 128
_S = 2
_NCH = 2560
_PKROWS = 2576
_ACC_N = 10240
_SLAB = _ACC_N // _NS
_D2 = 64


def _make_sc_agg(feature_split):
    mesh = plsc.VectorSubcoreMesh(core_axis_name="c", subcore_axis_name="s")
    cpw = _NCH // _NS if feature_split else _NCH // (_NC * _NS)
    half = _NCH // (_NC * _NS)
    nhalves = cpw // half

    @functools.partial(
        pl.kernel,
        out_type=jax.ShapeDtypeStruct((_NC, _ACC_N, _D2), jnp.float32),
        mesh=mesh,
        compiler_params=pltpu.CompilerParams(use_tc_tiling_on_sc=False),
        scratch_types=[
            pltpu.VMEM((half + 8, _CHUNK), jnp.int32),
            pltpu.VMEM((_S, _CHUNK), jnp.int32),
            pltpu.VMEM((_S, _CHUNK), jnp.int32),
            pltpu.VMEM((_S, _CHUNK, _D2), jnp.float32),
            pltpu.VMEM_SHARED((_ACC_N, _D2), jnp.float32),
            pltpu.VMEM_SHARED((_ACC_N, _D2), jnp.float32),
            pltpu.SemaphoreType.DMA,
        ],
    )
    def sc_agg(tab_hbm, pk_hbm, out_hbm, pk_v, src_v, dst_v, rows_v,
               tab_sh, acc_sh, sem_g):
        cid = lax.axis_index("c")
        sid = lax.axis_index("s")

        def zbody(i, carry):
            for k in range(_D2 // 16):
                rows_v[0, i, pl.ds(k * 16, 16)] = jnp.zeros((16,), jnp.float32)
            return carry

        lax.fori_loop(0, _CHUNK, zbody, 0)
        zbase = sid * _SLAB
        for t in range(_SLAB // _CHUNK):
            pltpu.sync_copy(rows_v.at[0],
                            acc_sh.at[pl.ds(zbase + t * _CHUNK, _CHUNK)])

        if feature_split:
            base = sid * cpw
            pltpu.sync_copy(tab_hbm.at[cid, pl.ds(zbase, _SLAB)],
                            tab_sh.at[pl.ds(zbase, _SLAB)])
        else:
            base = cid * (_NCH // _NC) + sid * cpw
            pltpu.sync_copy(tab_hbm.at[pl.ds(zbase, _SLAB)],
                            tab_sh.at[pl.ds(zbase, _SLAB)])
        plsc.subcore_barrier()

        def unpack(c, slot):
            for i in range(_CHUNK // 16):
                v = pk_v[c, pl.ds(i * 16, 16)]
                src_v[slot, pl.ds(i * 16, 16)] = v & 0xFFFF
                dst_v[slot, pl.ds(i * 16, 16)] = lax.shift_right_logical(v, 16)

        def fire(slot):
            return pltpu.async_copy(tab_sh.at[src_v.at[slot]],
                                    rows_v.at[slot], sem_g)

        def drain(slot):
            pltpu.make_async_copy(tab_sh.at[src_v.at[slot]],
                                  rows_v.at[slot], sem_g).wait()

        def gbody(u, carry):
            c = _S * u
            for slot in range(_S):
                drain(slot)
                pltpu.sync_copy(rows_v.at[slot],
                                acc_sh.at[dst_v.at[slot]], add=True)
                unpack(c + slot + _S, slot)
                fire(slot)
            return carry

        for h in range(nhalves):
            pltpu.sync_copy(pk_hbm.at[pl.ds(base + h * half, half + 8)], pk_v)
            for s in range(_S):
                unpack(s, s)
                fire(s)
            lax.fori_loop(0, half // _S, gbody, 0)
            for s in range(_S):
                drain(s)
        plsc.subcore_barrier()

        pltpu.sync_copy(acc_sh.at[pl.ds(zbase, _SLAB)],
                        out_hbm.at[cid, pl.ds(zbase, _SLAB)])

    return sc_agg


_BN = 1000


def _mm0_body(x_ref, w_ref, o_ref):
    r = jnp.dot(x_ref[...], w_ref[...], preferred_element_type=jnp.float32)
    o_ref[0] = r[:, :_D2]
    o_ref[1] = r[:, _D2:]


def _mm_mid_body(p_ref, b_ref, w_ref, o_ref):
    x = jnp.concatenate([p_ref[0], p_ref[1]], axis=1) + b_ref[...]
    x = jnp.maximum(x, 0.0)
    r = jnp.dot(x, w_ref[...], preferred_element_type=jnp.float32)
    o_ref[0] = r[:, :_D2]
    o_ref[1] = r[:, _D2:]


def _mm_last_body(p_ref, b_ref, w_ref, o_ref):
    x = jnp.concatenate([p_ref[0], p_ref[1]], axis=1) + b_ref[...]
    x = jnp.maximum(x, 0.0)
    o_ref[...] = jnp.dot(x, w_ref[...], preferred_element_type=jnp.float32)


def _final_body(p_ref, b_ref, o_ref):
    x = p_ref[0] + p_ref[1] + b_ref[...]
    m = jnp.max(x, axis=1, keepdims=True)
    s = x - m
    lse = jnp.log(jnp.sum(jnp.exp(s), axis=1, keepdims=True))
    o_ref[...] = s - lse


def _mm0(x, w):
    n, di = x.shape
    return pl.pallas_call(
        _mm0_body,
        grid=(n // _BN,),
        in_specs=[pl.BlockSpec((_BN, di), lambda i: (i, 0)),
                  pl.BlockSpec((di, 2 * _D2), lambda i: (0, 0))],
        out_specs=pl.BlockSpec((2, _BN, _D2), lambda i: (0, i, 0)),
        out_shape=jax.ShapeDtypeStruct((2, _ACC_N, _D2), jnp.float32),
    )(x, w)


def _mm_mid(p, b, w):
    di = 2 * _D2
    bn = 640
    return pl.pallas_call(
        _mm_mid_body,
        grid=(_ACC_N // bn,),
        in_specs=[pl.BlockSpec((2, bn, _D2), lambda i: (0, i, 0)),
                  pl.BlockSpec((1, di), lambda i: (0, 0)),
                  pl.BlockSpec((di, di), lambda i: (0, 0))],
        out_specs=pl.BlockSpec((2, bn, _D2), lambda i: (0, i, 0)),
        out_shape=jax.ShapeDtypeStruct((2, _ACC_N, _D2), jnp.float32),
    )(p, b.reshape(1, di), w)


def _mm_last(p, b, w):
    di = 2 * _D2
    bn = 640
    return pl.pallas_call(
        _mm_last_body,
        grid=(_ACC_N // bn,),
        in_specs=[pl.BlockSpec((2, bn, _D2), lambda i: (0, i, 0)),
                  pl.BlockSpec((1, di), lambda i: (0, 0)),
                  pl.BlockSpec((di, _D2), lambda i: (0, 0))],
        out_specs=pl.BlockSpec((bn, _D2), lambda i: (i, 0)),
        out_shape=jax.ShapeDtypeStruct((_ACC_N, _D2), jnp.float32),
    )(p, b.reshape(1, di), w)


def _final(p, b):
    do = b.shape[0]
    return pl.pallas_call(
        _final_body,
        grid=(_N // _BN,),
        in_specs=[pl.BlockSpec((2, _BN, do), lambda i: (0, i, 0)),
                  pl.BlockSpec((1, do), lambda i: (0, 0))],
        out_specs=pl.BlockSpec((_BN, do), lambda i: (i, 0)),
        out_shape=jax.ShapeDtypeStruct((_N, do), jnp.float32),
    )(p, b.reshape(1, do))


def kernel(features, edge_index, labels, mask, W0, b0, W1, b1, W2, b2):
    src = edge_index[0]
    dst = edge_index[1]
    npad = _PKROWS * _CHUNK - _E
    pk = jnp.concatenate([src | (dst << 16),
                          jnp.full((npad,), _N << 16, jnp.int32)]
                         ).reshape(_PKROWS, _CHUNK)

    agg_feat = _make_sc_agg(True)
    agg_edge = _make_sc_agg(False)

    g0 = _mm0(features, W0)
    s0 = agg_feat(g0, pk)
    g1 = _mm_mid(s0, b0, W1)
    s1 = agg_feat(g1, pk)
    g2 = _mm_last(s1, b1, W2)
    s2 = agg_edge(g2, pk)
    return _final(s2, b2)
